# baseline (speedup 1.0000x reference)
import functools

import jax
import jax.numpy as jnp
from jax.experimental import pallas as pl
from jax.experimental.pallas import tpu as pltpu

E = 8
R = 16
D = 1024
OUT = 1024
B = 2
S = 2048
IMG_START = 34
IMG_LEN = 576
SCALING = 32.0 / R

TILE = 1024
NIMG = 640
NC = 2
OC = OUT // NC

_DNT = (((1,), (1,)), ((), ()))


def _moe_tile_kernel(x_ref, w0_ref, b0_ref, wr_ref, br_ref, a2_ref,
                     bm_ref, out_ref, rout_ref, ec_ref,
                     wcat_ref, a2b_ref, xcat_ref, *, tiles_per_batch):
    t = pl.program_id(0)
    c = pl.program_id(1)
    tb = t % tiles_per_batch

    @pl.when(t == 0)
    def _():
        wcat_ref[pl.ds(c * OC, OC), :D] = w0_ref[...].astype(jnp.bfloat16)

        @pl.when(c == 0)
        def _():
            a2b_ref[...] = a2_ref[...].astype(jnp.bfloat16)
            for e in range(E):
                wcat_ref[:, D + e * R:D + (e + 1) * R] = (
                    bm_ref[e] * SCALING).astype(jnp.bfloat16)
            xcat_ref[NIMG:, D:] = jnp.zeros((TILE - NIMG, E * R),
                                            jnp.bfloat16)

    has_img = tb == 0

    @pl.when(c == 0)
    def _():
        x = x_ref[...]
        xb = x.astype(jnp.bfloat16)
        xcat_ref[:, :D] = xb

        @pl.when(has_img)
        def _():
            xs = x[:NIMG]
            logits = jax.lax.dot_general(
                xs, wr_ref[...], _DNT,
                preferred_element_type=jnp.float32) + br_ref[...]
            lmax = jnp.max(logits, axis=1, keepdims=True)
            ex = jnp.exp(logits - lmax)
            routing = ex / jnp.sum(ex, axis=1, keepdims=True)
            iota_e = jax.lax.broadcasted_iota(jnp.int32, (NIMG, E), 1)
            rmax = jnp.max(routing, axis=1, keepdims=True)
            idx = jnp.min(jnp.where(routing == rmax, iota_e, E), axis=1,
                          keepdims=True)
            y_hard = (iota_e == idx).astype(jnp.float32)
            rout_ref[0] = routing[IMG_START:IMG_START + IMG_LEN]
            ec = (y_hard - routing) + routing
            ec_ref[0] = ec[IMG_START:IMG_START + IMG_LEN]

            h = jax.lax.dot_general(xb[:NIMG], a2b_ref[...], _DNT,
                                    preferred_element_type=jnp.float32)
            col_e = jax.lax.broadcasted_iota(jnp.int32, (NIMG, E * R), 1) // R
            pos = jax.lax.broadcasted_iota(jnp.int32, (NIMG, 1), 0)
            is_img = jnp.logical_and(pos >= IMG_START,
                                     pos < IMG_START + IMG_LEN)
            hm = jnp.where(jnp.logical_and(col_e == idx, is_img), h, 0.0)
            xcat_ref[:NIMG, D:] = hm.astype(jnp.bfloat16)

    wc = wcat_ref[pl.ds(c * OC, OC), :]
    b0c = b0_ref[:, pl.ds(c * OC, OC)]

    @pl.when(has_img)
    def _():
        out = jax.lax.dot_general(xcat_ref[...], wc, _DNT,
                                  preferred_element_type=jnp.float32)
        out_ref[...] = out + b0c

    @pl.when(jnp.logical_not(has_img))
    def _():
        base = jax.lax.dot_general(xcat_ref[:, :D], wc[:, :D], _DNT,
                                   preferred_element_type=jnp.float32)
        out_ref[...] = base + b0c


@jax.jit
def kernel(x, W0, b0, Wr, br, A, Bm):
    xf = x.reshape(B * S, D)
    a2 = A.reshape(E * R, D)
    b0r = b0.reshape(1, OUT)
    brr = br.reshape(1, E)

    tiles_per_batch = S // TILE
    grid_t = (B * S) // TILE

    out, rout, ec = pl.pallas_call(
        functools.partial(_moe_tile_kernel, tiles_per_batch=tiles_per_batch),
        grid=(grid_t, NC),
        in_specs=[
            pl.BlockSpec((TILE, D), lambda t, c: (t, 0)),
            pl.BlockSpec((OC, D),
                         lambda t, c: (jnp.where(t == 0, c, NC - 1), 0)),
            pl.BlockSpec((1, OUT), lambda t, c: (0, 0)),
            pl.BlockSpec((E, D), lambda t, c: (0, 0)),
            pl.BlockSpec((1, E), lambda t, c: (0, 0)),
            pl.BlockSpec((E * R, D), lambda t, c: (0, 0)),
            pl.BlockSpec((E, OUT, R), lambda t, c: (0, 0, 0)),
        ],
        out_specs=[
            pl.BlockSpec((TILE, OC), lambda t, c: (t, c)),
            pl.BlockSpec((1, IMG_LEN, E),
                         lambda t, c: (t // (S // TILE), 0, 0)),
            pl.BlockSpec((1, IMG_LEN, E),
                         lambda t, c: (t // (S // TILE), 0, 0)),
        ],
        out_shape=[
            jax.ShapeDtypeStruct((B * S, OUT), jnp.float32),
            jax.ShapeDtypeStruct((B, IMG_LEN, E), jnp.float32),
            jax.ShapeDtypeStruct((B, IMG_LEN, E), jnp.float32),
        ],
        scratch_shapes=[
            pltpu.VMEM((OUT, D + E * R), jnp.bfloat16),
            pltpu.VMEM((E * R, D), jnp.bfloat16),
            pltpu.VMEM((TILE, D + E * R), jnp.bfloat16),
        ],
        compiler_params=pltpu.CompilerParams(
            dimension_semantics=("arbitrary", "arbitrary"),
        ),
    )(xf, W0, b0r, Wr, brr, a2, Bm)

    return (out.reshape(B, S, OUT), rout, ec)


# speedup vs baseline: 1.1158x; 1.1158x over previous
import functools

import jax
import jax.numpy as jnp
from jax.experimental import pallas as pl
from jax.experimental.pallas import tpu as pltpu

E = 8
R = 16
D = 1024
OUT = 1024
B = 2
S = 2048
IMG_START = 34
IMG_LEN = 576
SCALING = 32.0 / R

TILE = 1024
NIMG = 640

_DNT = (((1,), (1,)), ((), ()))


def _moe_tile_kernel(x_ref, w0_ref, b0_ref, wr_ref, br_ref, a2_ref,
                     bm_ref, out_ref, rout_ref, ec_ref,
                     wcat_ref, a2b_ref, xcat_ref, *, tiles_per_batch):
    t = pl.program_id(0)
    tb = t % tiles_per_batch

    @pl.when(t == 0)
    def _():
        wcat_ref[:, :D] = w0_ref[...]
        a2b_ref[...] = a2_ref[...].astype(jnp.bfloat16)
        for e in range(E):
            wcat_ref[:, D + e * R:D + (e + 1) * R] = (
                bm_ref[e] * SCALING).astype(jnp.bfloat16)
        xcat_ref[NIMG:, D:] = jnp.zeros((TILE - NIMG, E * R), jnp.bfloat16)

    x = x_ref[...]
    xb = x.astype(jnp.bfloat16)
    xcat_ref[:, :D] = xb

    has_img = tb == 0

    @pl.when(has_img)
    def _():
        xs = x[:NIMG]
        logits = jax.lax.dot_general(
            xs, wr_ref[...], _DNT,
            preferred_element_type=jnp.float32) + br_ref[...]
        lmax = jnp.max(logits, axis=1, keepdims=True)
        ex = jnp.exp(logits - lmax)
        routing = ex / jnp.sum(ex, axis=1, keepdims=True)
        iota_e = jax.lax.broadcasted_iota(jnp.int32, (NIMG, E), 1)
        rmax = jnp.max(routing, axis=1, keepdims=True)
        idx = jnp.min(jnp.where(routing == rmax, iota_e, E), axis=1,
                      keepdims=True)
        y_hard = (iota_e == idx).astype(jnp.float32)
        rout_ref[0] = routing[IMG_START:IMG_START + IMG_LEN]
        ec = (y_hard - routing) + routing
        ec_ref[0] = ec[IMG_START:IMG_START + IMG_LEN]

        h = jax.lax.dot_general(xb[:NIMG], a2b_ref[...], _DNT,
                                preferred_element_type=jnp.float32)
        col_e = jax.lax.broadcasted_iota(jnp.int32, (NIMG, E * R), 1) // R
        pos = jax.lax.broadcasted_iota(jnp.int32, (NIMG, 1), 0)
        is_img = jnp.logical_and(pos >= IMG_START, pos < IMG_START + IMG_LEN)
        hm = jnp.where(jnp.logical_and(col_e == idx, is_img), h, 0.0)
        xcat_ref[:NIMG, D:] = hm.astype(jnp.bfloat16)
        out = jax.lax.dot_general(xcat_ref[...], wcat_ref[...], _DNT,
                                  preferred_element_type=jnp.float32)
        out_ref[...] = out + b0_ref[...]

    @pl.when(jnp.logical_not(has_img))
    def _():
        base = jax.lax.dot_general(xcat_ref[:, :D], wcat_ref[:, :D], _DNT,
                                   preferred_element_type=jnp.float32)
        out_ref[...] = base + b0_ref[...]


@jax.jit
def kernel(x, W0, b0, Wr, br, A, Bm):
    xf = x.reshape(B * S, D)
    w0b = W0.astype(jnp.bfloat16)
    a2 = A.reshape(E * R, D)
    b0r = b0.reshape(1, OUT)
    brr = br.reshape(1, E)

    tiles_per_batch = S // TILE
    grid = (B * S) // TILE

    out, rout, ec = pl.pallas_call(
        functools.partial(_moe_tile_kernel, tiles_per_batch=tiles_per_batch),
        grid=(grid,),
        in_specs=[
            pl.BlockSpec((TILE, D), lambda t: (t, 0)),
            pl.BlockSpec((OUT, D), lambda t: (0, 0)),
            pl.BlockSpec((1, OUT), lambda t: (0, 0)),
            pl.BlockSpec((E, D), lambda t: (0, 0)),
            pl.BlockSpec((1, E), lambda t: (0, 0)),
            pl.BlockSpec((E * R, D), lambda t: (0, 0)),
            pl.BlockSpec((E, OUT, R), lambda t: (0, 0, 0)),
        ],
        out_specs=[
            pl.BlockSpec((TILE, OUT), lambda t: (t, 0)),
            pl.BlockSpec((1, IMG_LEN, E),
                         lambda t: (t // (S // TILE), 0, 0)),
            pl.BlockSpec((1, IMG_LEN, E),
                         lambda t: (t // (S // TILE), 0, 0)),
        ],
        out_shape=[
            jax.ShapeDtypeStruct((B * S, OUT), jnp.float32),
            jax.ShapeDtypeStruct((B, IMG_LEN, E), jnp.float32),
            jax.ShapeDtypeStruct((B, IMG_LEN, E), jnp.float32),
        ],
        scratch_shapes=[
            pltpu.VMEM((OUT, D + E * R), jnp.bfloat16),
            pltpu.VMEM((E * R, D), jnp.bfloat16),
            pltpu.VMEM((TILE, D + E * R), jnp.bfloat16),
        ],
        compiler_params=pltpu.CompilerParams(
            dimension_semantics=("arbitrary",),
        ),
    )(xf, w0b, b0r, Wr, brr, a2, Bm)

    return (out.reshape(B, S, OUT), rout, ec)


# speedup vs baseline: 1.2026x; 1.0778x over previous
import functools

import jax
import jax.numpy as jnp
from jax.experimental import pallas as pl
from jax.experimental.pallas import tpu as pltpu

E = 8
R = 16
D = 1024
OUT = 1024
B = 2
S = 2048
IMG_START = 34
IMG_LEN = 576
SCALING = 32.0 / R

TILE = 1024
NIMG = 640

_DNT = (((1,), (1,)), ((), ()))


def _moe_tile_kernel(x_ref, w0_ref, b0_ref, wr_ref, br_ref, a2_ref,
                     bm_ref, out_ref, rout_ref, ec_ref,
                     wcat_ref, a2b_ref, xcat_ref, *, tiles_per_batch):
    t = pl.program_id(0)
    tb = t % tiles_per_batch

    @pl.when(t == 0)
    def _():
        wcat_ref[:, :D] = w0_ref[...].astype(jnp.bfloat16)
        a2b_ref[...] = a2_ref[...].astype(jnp.bfloat16)
        for e in range(E):
            wcat_ref[:, D + e * R:D + (e + 1) * R] = (
                bm_ref[e] * SCALING).astype(jnp.bfloat16)
        xcat_ref[NIMG:, D:] = jnp.zeros((TILE - NIMG, E * R), jnp.bfloat16)

    x = x_ref[...]
    xb = x.astype(jnp.bfloat16)
    xcat_ref[:, :D] = xb

    has_img = tb == 0

    @pl.when(has_img)
    def _():
        xs = x[:NIMG]
        logits = jax.lax.dot_general(
            xs, wr_ref[...], _DNT,
            preferred_element_type=jnp.float32) + br_ref[...]
        lmax = jnp.max(logits, axis=1, keepdims=True)
        ex = jnp.exp(logits - lmax)
        routing = ex / jnp.sum(ex, axis=1, keepdims=True)
        iota_e = jax.lax.broadcasted_iota(jnp.int32, (NIMG, E), 1)
        rmax = jnp.max(routing, axis=1, keepdims=True)
        idx = jnp.min(jnp.where(routing == rmax, iota_e, E), axis=1,
                      keepdims=True)
        y_hard = (iota_e == idx).astype(jnp.float32)
        rout_ref[0] = routing[IMG_START:IMG_START + IMG_LEN]
        ec = (y_hard - routing) + routing
        ec_ref[0] = ec[IMG_START:IMG_START + IMG_LEN]

        h = jax.lax.dot_general(xb[:NIMG], a2b_ref[...], _DNT,
                                preferred_element_type=jnp.float32)
        col_e = jax.lax.broadcasted_iota(jnp.int32, (NIMG, E * R), 1) // R
        pos = jax.lax.broadcasted_iota(jnp.int32, (NIMG, 1), 0)
        is_img = jnp.logical_and(pos >= IMG_START, pos < IMG_START + IMG_LEN)
        hm = jnp.where(jnp.logical_and(col_e == idx, is_img), h, 0.0)
        xcat_ref[:NIMG, D:] = hm.astype(jnp.bfloat16)
        out = jax.lax.dot_general(xcat_ref[...], wcat_ref[...], _DNT,
                                  preferred_element_type=jnp.float32)
        out_ref[...] = out + b0_ref[...]

    @pl.when(jnp.logical_not(has_img))
    def _():
        base = jax.lax.dot_general(xcat_ref[:, :D], wcat_ref[:, :D], _DNT,
                                   preferred_element_type=jnp.float32)
        out_ref[...] = base + b0_ref[...]


@jax.jit
def kernel(x, W0, b0, Wr, br, A, Bm):
    xf = x.reshape(B * S, D)
    a2 = A.reshape(E * R, D)
    b0r = b0.reshape(1, OUT)
    brr = br.reshape(1, E)

    tiles_per_batch = S // TILE
    grid = (B * S) // TILE

    out, rout, ec = pl.pallas_call(
        functools.partial(_moe_tile_kernel, tiles_per_batch=tiles_per_batch),
        grid=(grid,),
        in_specs=[
            pl.BlockSpec((TILE, D), lambda t: (t, 0)),
            pl.BlockSpec((OUT, D), lambda t: (0, 0)),
            pl.BlockSpec((1, OUT), lambda t: (0, 0)),
            pl.BlockSpec((E, D), lambda t: (0, 0)),
            pl.BlockSpec((1, E), lambda t: (0, 0)),
            pl.BlockSpec((E * R, D), lambda t: (0, 0)),
            pl.BlockSpec((E, OUT, R), lambda t: (0, 0, 0)),
        ],
        out_specs=[
            pl.BlockSpec((TILE, OUT), lambda t: (t, 0)),
            pl.BlockSpec((1, IMG_LEN, E),
                         lambda t: (t // (S // TILE), 0, 0)),
            pl.BlockSpec((1, IMG_LEN, E),
                         lambda t: (t // (S // TILE), 0, 0)),
        ],
        out_shape=[
            jax.ShapeDtypeStruct((B * S, OUT), jnp.float32),
            jax.ShapeDtypeStruct((B, IMG_LEN, E), jnp.float32),
            jax.ShapeDtypeStruct((B, IMG_LEN, E), jnp.float32),
        ],
        scratch_shapes=[
            pltpu.VMEM((OUT, D + E * R), jnp.bfloat16),
            pltpu.VMEM((E * R, D), jnp.bfloat16),
            pltpu.VMEM((TILE, D + E * R), jnp.bfloat16),
        ],
        compiler_params=pltpu.CompilerParams(
            dimension_semantics=("arbitrary",),
        ),
    )(xf, W0, b0r, Wr, brr, a2, Bm)

    return (out.reshape(B, S, OUT), rout, ec)
